# baseline (speedup 1.0000x reference)
import jax
import jax.numpy as jnp
from jax import lax
from jax.experimental import pallas as pl
from jax.experimental.pallas import tpu as pltpu
from jax.experimental.pallas import tpu_sc as plsc

N = 10000
E = 320000
D = 128
C = 40
H = 8
K = 16
NC = 2
NS = 16
NW = NC * NS
EW = E // NW
B = 80
NCH = EW // B
NP = 10240
ROWS = NP // NS
RB = 400
NEG = 0.2


def _tc_a(x_ref, w1_ref, s1_ref, t1_ref, h_ref, ad_ref):
    h = jnp.dot(x_ref[:, :], w1_ref[:, :], preferred_element_type=jnp.float32)
    asrc = jnp.dot(h, s1_ref[:, :], preferred_element_type=jnp.float32)
    adst = jnp.dot(h, t1_ref[:, :], preferred_element_type=jnp.float32)
    z8 = jnp.zeros((RB, 8), jnp.float32)
    h_ref[:, :] = jnp.concatenate([h, asrc, z8], axis=1)
    ad_ref[:, :] = jnp.concatenate([adst, z8], axis=1)


def _make_sc_body(row_w, alpha_off, ex_off, spec):
    def body(src_hbm, dst_hbm, tab_hbm, adt_hbm, zo_hbm,
             out_hbm,
             idxs0, idxs1, idxs2, idxs3, idxd0, idxd1, idxd2, idxd3,
             tab0, tab1, ad0, ad1,
             sem_is0, sem_is1, sem_is2, sem_is3,
             sem_id0, sem_id1, sem_id2, sem_id3,
             sem_t0, sem_t1, sem_a0, sem_a1, sem_s0, sem_s1,
             out_acc):
        cid = lax.axis_index("c")
        sid = lax.axis_index("s")
        wid = sid * NC + cid

        idxs = (idxs0, idxs1, idxs2, idxs3)
        idxd = (idxd0, idxd1, idxd2, idxd3)
        tab = (tab0, tab1)
        ad = (ad0, ad1)
        sem_is = (sem_is0, sem_is1, sem_is2, sem_is3)
        sem_id = (sem_id0, sem_id1, sem_id2, sem_id3)
        sem_t = (sem_t0, sem_t1)
        sem_a = (sem_a0, sem_a1)
        sem_s = (sem_s0, sem_s1)

        r0 = sid * ROWS
        pltpu.sync_copy(zo_hbm.at[pl.ds(r0, ROWS)], out_acc.at[pl.ds(r0, ROWS)])
        plsc.subcore_barrier()

        ebase = wid * EW

        def issue_idx(j, s):
            pltpu.async_copy(src_hbm.at[pl.ds(ebase + j * B, B)], idxs[s],
                             sem_is[s])
            pltpu.async_copy(dst_hbm.at[pl.ds(ebase + j * B, B)], idxd[s],
                             sem_id[s])

        def wait_idx(j, s):
            pltpu.make_async_copy(src_hbm.at[pl.ds(ebase + j * B, B)],
                                  idxs[s], sem_is[s]).wait()
            pltpu.make_async_copy(dst_hbm.at[pl.ds(ebase + j * B, B)],
                                  idxd[s], sem_id[s]).wait()

        def issue_gathers(b, s):
            pltpu.async_copy(tab_hbm.at[idxs[s]], tab[b], sem_t[b])
            pltpu.async_copy(adt_hbm.at[idxd[s]], ad[b], sem_a[b])

        def wait_gathers(b, s):
            pltpu.make_async_copy(tab_hbm.at[idxs[s]], tab[b], sem_t[b]).wait()
            pltpu.make_async_copy(adt_hbm.at[idxd[s]], ad[b], sem_a[b]).wait()

        def compute(b):
            def edge_body(e, carry):
                a = tab[b][e, pl.ds(alpha_off, 16)] + ad[b][e, :]
                a = jnp.where(a >= 0.0, a, NEG * a)
                exv = jnp.exp(a)
                for lane, start in spec:
                    s_ = exv[lane]
                    tab[b][e, pl.ds(start, 16)] = tab[b][e, pl.ds(start, 16)] * s_
                tab[b][e, pl.ds(ex_off, 16)] = exv
                return carry
            lax.fori_loop(0, B, edge_body, 0, unroll=4)

        def step(i, k):
            cur, nxt = k % 2, (k + 1) % 2
            s_cur, s_nxt, s_pre, s_prev = k, (k + 1) % 4, (k + 2) % 4, (k - 1) % 4
            @pl.when(i >= 1)
            def _():
                pltpu.make_async_copy(
                    tab[nxt], out_acc.at[idxd[s_prev]], sem_s[nxt]).wait()
            @pl.when(i + 1 < NCH)
            def _():
                wait_idx(i + 1, s_nxt)
                issue_gathers(nxt, s_nxt)
            @pl.when(i + 2 < NCH)
            def _():
                issue_idx(i + 2, s_pre)
            wait_gathers(cur, s_cur)
            compute(cur)
            pltpu.async_copy(tab[cur], out_acc.at[idxd[s_cur]], sem_s[cur],
                             add=True)

        issue_idx(0, 0)
        wait_idx(0, 0)
        issue_gathers(0, 0)
        issue_idx(1, 1)

        def chunk_body(i, carry):
            for k in range(4):
                @pl.when(i % 4 == k)
                def _(k=k):
                    step(i, k)
            return carry

        lax.fori_loop(0, NCH, chunk_body, 0)

        last = (NCH - 1) % 2
        pltpu.make_async_copy(tab[last], out_acc.at[idxd[(NCH - 1) % 4]],
                              sem_s[last]).wait()
        plsc.subcore_barrier()

        pltpu.sync_copy(out_acc.at[pl.ds(r0, ROWS)],
                        out_hbm.at[cid, pl.ds(r0, ROWS)])

    return body


def _sc_edge_pass(src, dst, tab, adt, zo, row_w, alpha_off, ex_off, spec):
    mesh = plsc.VectorSubcoreMesh(core_axis_name="c", subcore_axis_name="s",
                                  num_cores=NC, num_subcores=NS)
    body = _make_sc_body(row_w, alpha_off, ex_off, spec)
    dma = pltpu.SemaphoreType.DMA
    return pl.kernel(
        body,
        out_type=jax.ShapeDtypeStruct((NC, NP, row_w), jnp.float32),
        mesh=mesh,
        compiler_params=pltpu.CompilerParams(use_tc_tiling_on_sc=False),
        scratch_types=[
            pltpu.VMEM((B,), jnp.int32),
            pltpu.VMEM((B,), jnp.int32),
            pltpu.VMEM((B,), jnp.int32),
            pltpu.VMEM((B,), jnp.int32),
            pltpu.VMEM((B,), jnp.int32),
            pltpu.VMEM((B,), jnp.int32),
            pltpu.VMEM((B,), jnp.int32),
            pltpu.VMEM((B,), jnp.int32),
            pltpu.VMEM((B, row_w), jnp.float32),
            pltpu.VMEM((B, row_w), jnp.float32),
            pltpu.VMEM((B, 16), jnp.float32),
            pltpu.VMEM((B, 16), jnp.float32),
            dma, dma, dma, dma, dma, dma, dma,
            dma, dma, dma, dma, dma, dma, dma,
            pltpu.VMEM_SHARED((NP, row_w), jnp.float32),
        ],
    )(src, dst, tab, adt, zo)


def _tc_c(o1_ref, h_ref, ad_ref, w2_ref, as2_ref, ad2_ref,
          b1_ref, p_ref, h2_ref, ado_ref):
    h1 = h_ref[:, 0:128]
    a = h_ref[:, 128:136] + ad_ref[:, 0:8]
    exs = jnp.exp(jnp.where(a >= 0.0, a, NEG * a))
    den = o1_ref[0, :, 128:136] + o1_ref[1, :, 128:136] + exs
    exs_e = jnp.dot(exs, p_ref[:, :], preferred_element_type=jnp.float32)
    den_e = jnp.dot(den, p_ref[:, :], preferred_element_type=jnp.float32)
    num = o1_ref[0, :, 0:128] + o1_ref[1, :, 0:128] + h1 * exs_e
    g = num / (den_e + 1e-16) + b1_ref[0, :][None, :]
    g = jnp.maximum(g, 0.0)
    h2 = jnp.dot(g, w2_ref[:, :], preferred_element_type=jnp.float32)
    asrc2 = jnp.sum(h2 * as2_ref[0, :][None, :], axis=1, keepdims=True)
    adst2 = jnp.sum(h2 * ad2_ref[0, :][None, :], axis=1, keepdims=True)
    z8 = jnp.zeros((RB, 8), jnp.float32)
    z15 = jnp.zeros((RB, 15), jnp.float32)
    h2_ref[:, :] = jnp.concatenate([h2, z8, asrc2, z15], axis=1)
    ado_ref[:, :] = jnp.concatenate([adst2, z15], axis=1)


def _tc_e(o2_ref, h2_ref, ad2_ref, b2_ref, out_ref):
    h2 = h2_ref[:, 0:40]
    a = h2_ref[:, 48:49] + ad2_ref[:, 0:1]
    exs = jnp.exp(jnp.where(a >= 0.0, a, NEG * a))
    den = o2_ref[0, :, 48:49] + o2_ref[1, :, 48:49] + exs
    num = o2_ref[0, :, 0:40] + o2_ref[1, :, 0:40] + h2 * exs
    o = num / (den + 1e-16) + b2_ref[0, :][None, :]
    m = jnp.max(o, axis=1, keepdims=True)
    z = o - m
    lse = jnp.log(jnp.sum(jnp.exp(z), axis=1, keepdims=True))
    out_ref[:, :] = z - lse


def kernel(x, edge_index, W1, att_src1, att_dst1, b1, W2, att_src2, att_dst2, b2):
    src = edge_index[0]
    dst = edge_index[1]

    eye8 = jnp.eye(H, dtype=jnp.float32)
    S1 = (att_src1[:, :, None] * eye8[:, None, :]).reshape(H * K, H)
    T1 = (att_dst1[:, :, None] * eye8[:, None, :]).reshape(H * K, H)
    P = jnp.repeat(eye8, K, axis=1)

    htab1, ad1 = pl.pallas_call(
        _tc_a,
        grid=(N // RB,),
        in_specs=[
            pl.BlockSpec((RB, D), lambda i: (i, 0)),
            pl.BlockSpec((D, D), lambda i: (0, 0)),
            pl.BlockSpec((D, H), lambda i: (0, 0)),
            pl.BlockSpec((D, H), lambda i: (0, 0)),
        ],
        out_specs=[
            pl.BlockSpec((RB, 144), lambda i: (i, 0)),
            pl.BlockSpec((RB, 16), lambda i: (i, 0)),
        ],
        out_shape=[
            jax.ShapeDtypeStruct((N, 144), jnp.float32),
            jax.ShapeDtypeStruct((N, 16), jnp.float32),
        ],
    )(x, W1, S1, T1)

    zo1 = jnp.zeros((NP, 144), jnp.float32)
    spec1 = [(h, 16 * h) for h in range(H)]
    out1p = _sc_edge_pass(src, dst, htab1, ad1, zo1,
                          row_w=144, alpha_off=128, ex_off=128, spec=spec1)

    htab2, ad2 = pl.pallas_call(
        _tc_c,
        grid=(N // RB,),
        in_specs=[
            pl.BlockSpec((NC, RB, 144), lambda i: (0, i, 0)),
            pl.BlockSpec((RB, 144), lambda i: (i, 0)),
            pl.BlockSpec((RB, 16), lambda i: (i, 0)),
            pl.BlockSpec((D, C), lambda i: (0, 0)),
            pl.BlockSpec((1, C), lambda i: (0, 0)),
            pl.BlockSpec((1, C), lambda i: (0, 0)),
            pl.BlockSpec((1, D), lambda i: (0, 0)),
            pl.BlockSpec((H, D), lambda i: (0, 0)),
        ],
        out_specs=[
            pl.BlockSpec((RB, 64), lambda i: (i, 0)),
            pl.BlockSpec((RB, 16), lambda i: (i, 0)),
        ],
        out_shape=[
            jax.ShapeDtypeStruct((N, 64), jnp.float32),
            jax.ShapeDtypeStruct((N, 16), jnp.float32),
        ],
    )(out1p, htab1, ad1, W2, att_src2, att_dst2, b1.reshape(1, D), P)

    zo2 = jnp.zeros((NP, 64), jnp.float32)
    spec2 = [(0, 0), (0, 16), (0, 32)]
    out2p = _sc_edge_pass(src, dst, htab2, ad2, zo2,
                          row_w=64, alpha_off=48, ex_off=48, spec=spec2)

    out = pl.pallas_call(
        _tc_e,
        grid=(N // RB,),
        in_specs=[
            pl.BlockSpec((NC, RB, 64), lambda i: (0, i, 0)),
            pl.BlockSpec((RB, 64), lambda i: (i, 0)),
            pl.BlockSpec((RB, 16), lambda i: (i, 0)),
            pl.BlockSpec((1, C), lambda i: (0, 0)),
        ],
        out_specs=pl.BlockSpec((RB, C), lambda i: (i, 0)),
        out_shape=jax.ShapeDtypeStruct((N, C), jnp.float32),
    )(out2p, htab2, ad2, b2.reshape(1, C))

    return out


# speedup vs baseline: 1.5531x; 1.5531x over previous
import jax
import jax.numpy as jnp
from jax import lax
from jax.experimental import pallas as pl
from jax.experimental.pallas import tpu as pltpu
from jax.experimental.pallas import tpu_sc as plsc

N = 10000
E = 320000
D = 128
C = 40
H = 8
K = 16
NC = 2
NS = 16
NW = NC * NS
EW = E // NW
B = 80
NCH = EW // B
NP = 10240
ROWS = NP // NS
RB = 400
NEG = 0.2


def _tc_a(x_ref, w1_ref, s1_ref, t1_ref, h_ref, ad_ref):
    h = jnp.dot(x_ref[:, :], w1_ref[:, :], preferred_element_type=jnp.float32)
    asrc = jnp.dot(h, s1_ref[:, :], preferred_element_type=jnp.float32)
    adst = jnp.dot(h, t1_ref[:, :], preferred_element_type=jnp.float32)
    z8 = jnp.zeros((RB, 8), jnp.float32)
    h_ref[:, :] = jnp.concatenate([h, asrc, z8], axis=1)
    ad_ref[:, :] = jnp.concatenate([adst, z8], axis=1)


def _make_sc_body(row_w, alpha_off, ex_off, spec):
    def body(src_hbm, dst_hbm, tab_hbm, adt_hbm, zo_hbm,
             out_hbm,
             idxs0, idxs1, idxs2, idxs3, idxd0, idxd1, idxd2, idxd3,
             tab0, tab1, ad0, ad1,
             sem_is0, sem_is1, sem_is2, sem_is3,
             sem_id0, sem_id1, sem_id2, sem_id3,
             sem_t0, sem_t1, sem_a0, sem_a1, sem_s0, sem_s1,
             out_acc):
        cid = lax.axis_index("c")
        sid = lax.axis_index("s")
        wid = sid * NC + cid

        idxs = (idxs0, idxs1, idxs2, idxs3)
        idxd = (idxd0, idxd1, idxd2, idxd3)
        tab = (tab0, tab1)
        ad = (ad0, ad1)
        sem_is = (sem_is0, sem_is1, sem_is2, sem_is3)
        sem_id = (sem_id0, sem_id1, sem_id2, sem_id3)
        sem_t = (sem_t0, sem_t1)
        sem_a = (sem_a0, sem_a1)
        sem_s = (sem_s0, sem_s1)

        r0 = sid * ROWS
        pltpu.sync_copy(zo_hbm.at[pl.ds(r0, ROWS)], out_acc.at[pl.ds(r0, ROWS)])
        plsc.subcore_barrier()

        ebase = wid * EW

        def issue_idx(j, s):
            pltpu.async_copy(src_hbm.at[pl.ds(ebase + j * B, B)], idxs[s],
                             sem_is[s])
            pltpu.async_copy(dst_hbm.at[pl.ds(ebase + j * B, B)], idxd[s],
                             sem_id[s])

        def wait_idx(j, s):
            pltpu.make_async_copy(src_hbm.at[pl.ds(ebase + j * B, B)],
                                  idxs[s], sem_is[s]).wait()
            pltpu.make_async_copy(dst_hbm.at[pl.ds(ebase + j * B, B)],
                                  idxd[s], sem_id[s]).wait()

        def issue_gathers(b, s):
            pltpu.async_copy(tab_hbm.at[idxs[s]], tab[b], sem_t[b])
            pltpu.async_copy(adt_hbm.at[idxd[s]], ad[b], sem_a[b])

        def wait_gathers(b, s):
            pltpu.make_async_copy(tab_hbm.at[idxs[s]], tab[b], sem_t[b]).wait()
            pltpu.make_async_copy(adt_hbm.at[idxd[s]], ad[b], sem_a[b]).wait()

        def compute(b):
            @plsc.parallel_loop(0, B, step=1, unroll=4)
            def edge_body(e):
                a = tab[b][e, pl.ds(alpha_off, 16)] + ad[b][e, :]
                a = jnp.where(a >= 0.0, a, NEG * a)
                exv = jnp.exp(a)
                for lane, start in spec:
                    s_ = exv[lane]
                    tab[b][e, pl.ds(start, 16)] = tab[b][e, pl.ds(start, 16)] * s_
                tab[b][e, pl.ds(ex_off, 16)] = exv

        def step(i, k):
            cur, nxt = k % 2, (k + 1) % 2
            s_cur, s_nxt, s_pre, s_prev = k, (k + 1) % 4, (k + 2) % 4, (k - 1) % 4
            @pl.when(i >= 1)
            def _():
                pltpu.make_async_copy(
                    tab[nxt], out_acc.at[idxd[s_prev]], sem_s[nxt]).wait()
            @pl.when(i + 1 < NCH)
            def _():
                wait_idx(i + 1, s_nxt)
                issue_gathers(nxt, s_nxt)
            @pl.when(i + 2 < NCH)
            def _():
                issue_idx(i + 2, s_pre)
            wait_gathers(cur, s_cur)
            compute(cur)
            pltpu.async_copy(tab[cur], out_acc.at[idxd[s_cur]], sem_s[cur],
                             add=True)

        issue_idx(0, 0)
        wait_idx(0, 0)
        issue_gathers(0, 0)
        issue_idx(1, 1)

        def chunk_body(i, carry):
            for k in range(4):
                @pl.when(i % 4 == k)
                def _(k=k):
                    step(i, k)
            return carry

        lax.fori_loop(0, NCH, chunk_body, 0)

        last = (NCH - 1) % 2
        pltpu.make_async_copy(tab[last], out_acc.at[idxd[(NCH - 1) % 4]],
                              sem_s[last]).wait()
        plsc.subcore_barrier()

        pltpu.sync_copy(out_acc.at[pl.ds(r0, ROWS)],
                        out_hbm.at[cid, pl.ds(r0, ROWS)])

    return body


def _sc_edge_pass(src, dst, tab, adt, zo, row_w, alpha_off, ex_off, spec):
    mesh = plsc.VectorSubcoreMesh(core_axis_name="c", subcore_axis_name="s",
                                  num_cores=NC, num_subcores=NS)
    body = _make_sc_body(row_w, alpha_off, ex_off, spec)
    dma = pltpu.SemaphoreType.DMA
    return pl.kernel(
        body,
        out_type=jax.ShapeDtypeStruct((NC, NP, row_w), jnp.float32),
        mesh=mesh,
        compiler_params=pltpu.CompilerParams(use_tc_tiling_on_sc=False),
        scratch_types=[
            pltpu.VMEM((B,), jnp.int32),
            pltpu.VMEM((B,), jnp.int32),
            pltpu.VMEM((B,), jnp.int32),
            pltpu.VMEM((B,), jnp.int32),
            pltpu.VMEM((B,), jnp.int32),
            pltpu.VMEM((B,), jnp.int32),
            pltpu.VMEM((B,), jnp.int32),
            pltpu.VMEM((B,), jnp.int32),
            pltpu.VMEM((B, row_w), jnp.float32),
            pltpu.VMEM((B, row_w), jnp.float32),
            pltpu.VMEM((B, 16), jnp.float32),
            pltpu.VMEM((B, 16), jnp.float32),
            dma, dma, dma, dma, dma, dma, dma,
            dma, dma, dma, dma, dma, dma, dma,
            pltpu.VMEM_SHARED((NP, row_w), jnp.float32),
        ],
    )(src, dst, tab, adt, zo)


def _tc_c(o1_ref, h_ref, ad_ref, w2_ref, as2_ref, ad2_ref,
          b1_ref, p_ref, h2_ref, ado_ref):
    h1 = h_ref[:, 0:128]
    a = h_ref[:, 128:136] + ad_ref[:, 0:8]
    exs = jnp.exp(jnp.where(a >= 0.0, a, NEG * a))
    den = o1_ref[0, :, 128:136] + o1_ref[1, :, 128:136] + exs
    exs_e = jnp.dot(exs, p_ref[:, :], preferred_element_type=jnp.float32)
    den_e = jnp.dot(den, p_ref[:, :], preferred_element_type=jnp.float32)
    num = o1_ref[0, :, 0:128] + o1_ref[1, :, 0:128] + h1 * exs_e
    g = num / (den_e + 1e-16) + b1_ref[0, :][None, :]
    g = jnp.maximum(g, 0.0)
    h2 = jnp.dot(g, w2_ref[:, :], preferred_element_type=jnp.float32)
    asrc2 = jnp.sum(h2 * as2_ref[0, :][None, :], axis=1, keepdims=True)
    adst2 = jnp.sum(h2 * ad2_ref[0, :][None, :], axis=1, keepdims=True)
    z8 = jnp.zeros((RB, 8), jnp.float32)
    z15 = jnp.zeros((RB, 15), jnp.float32)
    h2_ref[:, :] = jnp.concatenate([h2, z8, asrc2, z15], axis=1)
    ado_ref[:, :] = jnp.concatenate([adst2, z15], axis=1)


def _tc_e(o2_ref, h2_ref, ad2_ref, b2_ref, out_ref):
    h2 = h2_ref[:, 0:40]
    a = h2_ref[:, 48:49] + ad2_ref[:, 0:1]
    exs = jnp.exp(jnp.where(a >= 0.0, a, NEG * a))
    den = o2_ref[0, :, 48:49] + o2_ref[1, :, 48:49] + exs
    num = o2_ref[0, :, 0:40] + o2_ref[1, :, 0:40] + h2 * exs
    o = num / (den + 1e-16) + b2_ref[0, :][None, :]
    m = jnp.max(o, axis=1, keepdims=True)
    z = o - m
    lse = jnp.log(jnp.sum(jnp.exp(z), axis=1, keepdims=True))
    out_ref[:, :] = z - lse


def kernel(x, edge_index, W1, att_src1, att_dst1, b1, W2, att_src2, att_dst2, b2):
    src = edge_index[0]
    dst = edge_index[1]

    eye8 = jnp.eye(H, dtype=jnp.float32)
    S1 = (att_src1[:, :, None] * eye8[:, None, :]).reshape(H * K, H)
    T1 = (att_dst1[:, :, None] * eye8[:, None, :]).reshape(H * K, H)
    P = jnp.repeat(eye8, K, axis=1)

    htab1, ad1 = pl.pallas_call(
        _tc_a,
        grid=(N // RB,),
        in_specs=[
            pl.BlockSpec((RB, D), lambda i: (i, 0)),
            pl.BlockSpec((D, D), lambda i: (0, 0)),
            pl.BlockSpec((D, H), lambda i: (0, 0)),
            pl.BlockSpec((D, H), lambda i: (0, 0)),
        ],
        out_specs=[
            pl.BlockSpec((RB, 144), lambda i: (i, 0)),
            pl.BlockSpec((RB, 16), lambda i: (i, 0)),
        ],
        out_shape=[
            jax.ShapeDtypeStruct((N, 144), jnp.float32),
            jax.ShapeDtypeStruct((N, 16), jnp.float32),
        ],
    )(x, W1, S1, T1)

    zo1 = jnp.zeros((NP, 144), jnp.float32)
    spec1 = [(h, 16 * h) for h in range(H)]
    out1p = _sc_edge_pass(src, dst, htab1, ad1, zo1,
                          row_w=144, alpha_off=128, ex_off=128, spec=spec1)

    htab2, ad2 = pl.pallas_call(
        _tc_c,
        grid=(N // RB,),
        in_specs=[
            pl.BlockSpec((NC, RB, 144), lambda i: (0, i, 0)),
            pl.BlockSpec((RB, 144), lambda i: (i, 0)),
            pl.BlockSpec((RB, 16), lambda i: (i, 0)),
            pl.BlockSpec((D, C), lambda i: (0, 0)),
            pl.BlockSpec((1, C), lambda i: (0, 0)),
            pl.BlockSpec((1, C), lambda i: (0, 0)),
            pl.BlockSpec((1, D), lambda i: (0, 0)),
            pl.BlockSpec((H, D), lambda i: (0, 0)),
        ],
        out_specs=[
            pl.BlockSpec((RB, 64), lambda i: (i, 0)),
            pl.BlockSpec((RB, 16), lambda i: (i, 0)),
        ],
        out_shape=[
            jax.ShapeDtypeStruct((N, 64), jnp.float32),
            jax.ShapeDtypeStruct((N, 16), jnp.float32),
        ],
    )(out1p, htab1, ad1, W2, att_src2, att_dst2, b1.reshape(1, D), P)

    zo2 = jnp.zeros((NP, 64), jnp.float32)
    spec2 = [(0, 0), (0, 16), (0, 32)]
    out2p = _sc_edge_pass(src, dst, htab2, ad2, zo2,
                          row_w=64, alpha_off=48, ex_off=48, spec=spec2)

    out = pl.pallas_call(
        _tc_e,
        grid=(N // RB,),
        in_specs=[
            pl.BlockSpec((NC, RB, 64), lambda i: (0, i, 0)),
            pl.BlockSpec((RB, 64), lambda i: (i, 0)),
            pl.BlockSpec((RB, 16), lambda i: (i, 0)),
            pl.BlockSpec((1, C), lambda i: (0, 0)),
        ],
        out_specs=pl.BlockSpec((RB, C), lambda i: (i, 0)),
        out_shape=jax.ShapeDtypeStruct((N, C), jnp.float32),
    )(out2p, htab2, ad2, b2.reshape(1, C))

    return out


# speedup vs baseline: 1.6736x; 1.0776x over previous
import jax
import jax.numpy as jnp
from jax import lax
from jax.experimental import pallas as pl
from jax.experimental.pallas import tpu as pltpu
from jax.experimental.pallas import tpu_sc as plsc

N = 10000
E = 320000
D = 128
C = 40
H = 8
K = 16
NC = 2
NS = 16
NW = NC * NS
EW = E // NW
B = 80
NCH = EW // B
NP = 10240
ROWS = NP // NS
RB = 2000
NEG = 0.2


def _tc_a(x_ref, w1_ref, s1_ref, t1_ref, h_ref, ad_ref):
    h = jnp.dot(x_ref[:, :], w1_ref[:, :], preferred_element_type=jnp.float32)
    asrc = jnp.dot(h, s1_ref[:, :], preferred_element_type=jnp.float32)
    adst = jnp.dot(h, t1_ref[:, :], preferred_element_type=jnp.float32)
    z8 = jnp.zeros((RB, 8), jnp.float32)
    h_ref[:, :] = jnp.concatenate([h, asrc, z8], axis=1)
    ad_ref[:, :] = jnp.concatenate([adst, z8], axis=1)


def _make_sc_body(row_w, alpha_off, ex_off, spec):
    def body(src_hbm, dst_hbm, tab_hbm, adt_hbm, zo_hbm,
             out_hbm,
             idxs0, idxs1, idxs2, idxs3, idxd0, idxd1, idxd2, idxd3,
             tab0, tab1, ad0, ad1,
             sem_is0, sem_is1, sem_is2, sem_is3,
             sem_id0, sem_id1, sem_id2, sem_id3,
             sem_t0, sem_t1, sem_a0, sem_a1, sem_s0, sem_s1,
             out_acc):
        cid = lax.axis_index("c")
        sid = lax.axis_index("s")
        wid = sid * NC + cid

        idxs = (idxs0, idxs1, idxs2, idxs3)
        idxd = (idxd0, idxd1, idxd2, idxd3)
        tab = (tab0, tab1)
        ad = (ad0, ad1)
        sem_is = (sem_is0, sem_is1, sem_is2, sem_is3)
        sem_id = (sem_id0, sem_id1, sem_id2, sem_id3)
        sem_t = (sem_t0, sem_t1)
        sem_a = (sem_a0, sem_a1)
        sem_s = (sem_s0, sem_s1)

        r0 = sid * ROWS
        pltpu.sync_copy(zo_hbm.at[pl.ds(r0, ROWS)], out_acc.at[pl.ds(r0, ROWS)])
        plsc.subcore_barrier()

        ebase = wid * EW

        def issue_idx(j, s):
            pltpu.async_copy(src_hbm.at[pl.ds(ebase + j * B, B)], idxs[s],
                             sem_is[s])
            pltpu.async_copy(dst_hbm.at[pl.ds(ebase + j * B, B)], idxd[s],
                             sem_id[s])

        def wait_idx(j, s):
            pltpu.make_async_copy(src_hbm.at[pl.ds(ebase + j * B, B)],
                                  idxs[s], sem_is[s]).wait()
            pltpu.make_async_copy(dst_hbm.at[pl.ds(ebase + j * B, B)],
                                  idxd[s], sem_id[s]).wait()

        def issue_gathers(b, s):
            pltpu.async_copy(tab_hbm.at[idxs[s]], tab[b], sem_t[b])
            pltpu.async_copy(adt_hbm.at[idxd[s]], ad[b], sem_a[b])

        def wait_gathers(b, s):
            pltpu.make_async_copy(tab_hbm.at[idxs[s]], tab[b], sem_t[b]).wait()
            pltpu.make_async_copy(adt_hbm.at[idxd[s]], ad[b], sem_a[b]).wait()

        dup_idx = lax.rem(lax.iota(jnp.int32, 16), jnp.int32(8))

        def compute(b):
            @plsc.parallel_loop(0, B, step=1, unroll=4)
            def edge_body(e):
                a = tab[b][e, pl.ds(alpha_off, 16)] + ad[b][e, :]
                a = jnp.maximum(a, NEG * a)
                exv = jnp.exp(a)
                if spec == "dup8":
                    exd = exv.at[dup_idx].get(mode="promise_in_bounds")
                    for j in range(8):
                        tab[b][e, pl.ds(16 * j, 16)] = (
                            tab[b][e, pl.ds(16 * j, 16)] * exd)
                else:
                    for lane, start in spec:
                        s_ = exv[lane]
                        tab[b][e, pl.ds(start, 16)] = (
                            tab[b][e, pl.ds(start, 16)] * s_)
                tab[b][e, pl.ds(ex_off, 16)] = exv

        def step(i, k):
            cur, nxt = k % 2, (k + 1) % 2
            s_cur, s_nxt, s_pre, s_prev = k, (k + 1) % 4, (k + 2) % 4, (k - 1) % 4
            @pl.when(i >= 1)
            def _():
                pltpu.make_async_copy(
                    tab[nxt], out_acc.at[idxd[s_prev]], sem_s[nxt]).wait()
            @pl.when(i + 1 < NCH)
            def _():
                wait_idx(i + 1, s_nxt)
                issue_gathers(nxt, s_nxt)
            @pl.when(i + 2 < NCH)
            def _():
                issue_idx(i + 2, s_pre)
            wait_gathers(cur, s_cur)
            compute(cur)
            pltpu.async_copy(tab[cur], out_acc.at[idxd[s_cur]], sem_s[cur],
                             add=True)

        issue_idx(0, 0)
        wait_idx(0, 0)
        issue_gathers(0, 0)
        issue_idx(1, 1)

        def chunk_body(i, carry):
            for k in range(4):
                @pl.when(i % 4 == k)
                def _(k=k):
                    step(i, k)
            return carry

        lax.fori_loop(0, NCH, chunk_body, 0)

        last = (NCH - 1) % 2
        pltpu.make_async_copy(tab[last], out_acc.at[idxd[(NCH - 1) % 4]],
                              sem_s[last]).wait()
        plsc.subcore_barrier()

        pltpu.sync_copy(out_acc.at[pl.ds(r0, ROWS)],
                        out_hbm.at[cid, pl.ds(r0, ROWS)])

    return body


def _sc_edge_pass(src, dst, tab, adt, zo, row_w, alpha_off, ex_off, spec):
    mesh = plsc.VectorSubcoreMesh(core_axis_name="c", subcore_axis_name="s",
                                  num_cores=NC, num_subcores=NS)
    body = _make_sc_body(row_w, alpha_off, ex_off, spec)
    dma = pltpu.SemaphoreType.DMA
    return pl.kernel(
        body,
        out_type=jax.ShapeDtypeStruct((NC, NP, row_w), jnp.float32),
        mesh=mesh,
        compiler_params=pltpu.CompilerParams(use_tc_tiling_on_sc=False),
        scratch_types=[
            pltpu.VMEM((B,), jnp.int32),
            pltpu.VMEM((B,), jnp.int32),
            pltpu.VMEM((B,), jnp.int32),
            pltpu.VMEM((B,), jnp.int32),
            pltpu.VMEM((B,), jnp.int32),
            pltpu.VMEM((B,), jnp.int32),
            pltpu.VMEM((B,), jnp.int32),
            pltpu.VMEM((B,), jnp.int32),
            pltpu.VMEM((B, row_w), jnp.float32),
            pltpu.VMEM((B, row_w), jnp.float32),
            pltpu.VMEM((B, 16), jnp.float32),
            pltpu.VMEM((B, 16), jnp.float32),
            dma, dma, dma, dma, dma, dma, dma,
            dma, dma, dma, dma, dma, dma, dma,
            pltpu.VMEM_SHARED((NP, row_w), jnp.float32),
        ],
    )(src, dst, tab, adt, zo)


def _tc_c(o1_ref, h_ref, ad_ref, w2_ref, as2_ref, ad2_ref,
          b1_ref, p_ref, h2_ref, ado_ref):
    h1 = h_ref[:, 0:128]
    a = h_ref[:, 128:136] + ad_ref[:, 0:8]
    exs = jnp.exp(jnp.where(a >= 0.0, a, NEG * a))
    den = o1_ref[0, :, 128:136] + o1_ref[1, :, 128:136] + exs
    exs_e = jnp.dot(exs, p_ref[:, :], preferred_element_type=jnp.float32)
    den_e = jnp.dot(den, p_ref[:, :], preferred_element_type=jnp.float32)
    num = o1_ref[0, :, 0:128] + o1_ref[1, :, 0:128] + h1 * exs_e
    g = num / (den_e + 1e-16) + b1_ref[0, :][None, :]
    g = jnp.maximum(g, 0.0)
    h2 = jnp.dot(g, w2_ref[:, :], preferred_element_type=jnp.float32)
    asrc2 = jnp.sum(h2 * as2_ref[0, :][None, :], axis=1, keepdims=True)
    adst2 = jnp.sum(h2 * ad2_ref[0, :][None, :], axis=1, keepdims=True)
    z8 = jnp.zeros((RB, 8), jnp.float32)
    z15 = jnp.zeros((RB, 15), jnp.float32)
    h2_ref[:, :] = jnp.concatenate([h2, z8, asrc2, z15], axis=1)
    ado_ref[:, :] = jnp.concatenate([adst2, z15], axis=1)


def _tc_e(o2_ref, h2_ref, ad2_ref, b2_ref, out_ref):
    h2 = h2_ref[:, 0:40]
    a = h2_ref[:, 48:49] + ad2_ref[:, 0:1]
    exs = jnp.exp(jnp.where(a >= 0.0, a, NEG * a))
    den = o2_ref[0, :, 48:49] + o2_ref[1, :, 48:49] + exs
    num = o2_ref[0, :, 0:40] + o2_ref[1, :, 0:40] + h2 * exs
    o = num / (den + 1e-16) + b2_ref[0, :][None, :]
    m = jnp.max(o, axis=1, keepdims=True)
    z = o - m
    lse = jnp.log(jnp.sum(jnp.exp(z), axis=1, keepdims=True))
    out_ref[:, :] = z - lse


def kernel(x, edge_index, W1, att_src1, att_dst1, b1, W2, att_src2, att_dst2, b2):
    src = edge_index[0]
    dst = edge_index[1]

    eye8 = jnp.eye(H, dtype=jnp.float32)
    S1 = (att_src1[:, :, None] * eye8[:, None, :]).reshape(H * K, H)
    T1 = (att_dst1[:, :, None] * eye8[:, None, :]).reshape(H * K, H)
    j = jnp.arange(H * K)
    pvec = (j % H) * K + j // H
    W1t = W1[:, pvec]
    S1t = S1[pvec, :]
    T1t = T1[pvec, :]
    W2t = W2[pvec, :]
    b1t = b1[pvec]
    Pt = (j[None, :] % H == jnp.arange(H)[:, None]).astype(jnp.float32)

    htab1, ad1 = pl.pallas_call(
        _tc_a,
        grid=(N // RB,),
        in_specs=[
            pl.BlockSpec((RB, D), lambda i: (i, 0)),
            pl.BlockSpec((D, D), lambda i: (0, 0)),
            pl.BlockSpec((D, H), lambda i: (0, 0)),
            pl.BlockSpec((D, H), lambda i: (0, 0)),
        ],
        out_specs=[
            pl.BlockSpec((RB, 144), lambda i: (i, 0)),
            pl.BlockSpec((RB, 16), lambda i: (i, 0)),
        ],
        out_shape=[
            jax.ShapeDtypeStruct((N, 144), jnp.float32),
            jax.ShapeDtypeStruct((N, 16), jnp.float32),
        ],
    )(x, W1t, S1t, T1t)

    zo1 = jnp.zeros((NP, 144), jnp.float32)
    out1p = _sc_edge_pass(src, dst, htab1, ad1, zo1,
                          row_w=144, alpha_off=128, ex_off=128, spec="dup8")

    htab2, ad2 = pl.pallas_call(
        _tc_c,
        grid=(N // RB,),
        in_specs=[
            pl.BlockSpec((NC, RB, 144), lambda i: (0, i, 0)),
            pl.BlockSpec((RB, 144), lambda i: (i, 0)),
            pl.BlockSpec((RB, 16), lambda i: (i, 0)),
            pl.BlockSpec((D, C), lambda i: (0, 0)),
            pl.BlockSpec((1, C), lambda i: (0, 0)),
            pl.BlockSpec((1, C), lambda i: (0, 0)),
            pl.BlockSpec((1, D), lambda i: (0, 0)),
            pl.BlockSpec((H, D), lambda i: (0, 0)),
        ],
        out_specs=[
            pl.BlockSpec((RB, 64), lambda i: (i, 0)),
            pl.BlockSpec((RB, 16), lambda i: (i, 0)),
        ],
        out_shape=[
            jax.ShapeDtypeStruct((N, 64), jnp.float32),
            jax.ShapeDtypeStruct((N, 16), jnp.float32),
        ],
    )(out1p, htab1, ad1, W2t, att_src2, att_dst2, b1t.reshape(1, D), Pt)

    zo2 = jnp.zeros((NP, 64), jnp.float32)
    spec2 = [(0, 0), (0, 16), (0, 32)]
    out2p = _sc_edge_pass(src, dst, htab2, ad2, zo2,
                          row_w=64, alpha_off=48, ex_off=48, spec=spec2)

    out = pl.pallas_call(
        _tc_e,
        grid=(N // RB,),
        in_specs=[
            pl.BlockSpec((NC, RB, 64), lambda i: (0, i, 0)),
            pl.BlockSpec((RB, 64), lambda i: (i, 0)),
            pl.BlockSpec((RB, 16), lambda i: (i, 0)),
            pl.BlockSpec((1, C), lambda i: (0, 0)),
        ],
        out_specs=pl.BlockSpec((RB, C), lambda i: (i, 0)),
        out_shape=jax.ShapeDtypeStruct((N, C), jnp.float32),
    )(out2p, htab2, ad2, b2.reshape(1, C))

    return out


# speedup vs baseline: 1.7813x; 1.0644x over previous
import jax
import jax.numpy as jnp
from jax import lax
from jax.experimental import pallas as pl
from jax.experimental.pallas import tpu as pltpu
from jax.experimental.pallas import tpu_sc as plsc

N = 10000
E = 320000
D = 128
C = 40
H = 8
K = 16
NC = 2
NS = 16
NW = NC * NS
EW = E // NW
B = 80
NCH = EW // B
NP = 10112
ROWS = NP // NS
RB = 2000
NEG = 0.2


def _tc_a(x_ref, w1_ref, s1_ref, t1_ref, h_ref, ad_ref):
    h = jnp.dot(x_ref[:, :], w1_ref[:, :], preferred_element_type=jnp.float32)
    asrc = jnp.dot(h, s1_ref[:, :], preferred_element_type=jnp.float32)
    adst = jnp.dot(h, t1_ref[:, :], preferred_element_type=jnp.float32)
    z8 = jnp.zeros((RB, 8), jnp.float32)
    h_ref[:, :] = jnp.concatenate([h, asrc, z8], axis=1)
    ad_ref[:, :] = jnp.concatenate([adst, z8], axis=1)


def _make_sc_body(row_w, alpha_off, ex_off, spec):
    def body(src_hbm, dst_hbm, tab_hbm, adt_hbm, zo_hbm,
             out_hbm,
             idxs0, idxs1, idxs2, idxs3, idxd0, idxd1, idxd2, idxd3,
             tab0, tab1, tab2, ad0, ad1, ad2,
             sem_is0, sem_is1, sem_is2, sem_is3,
             sem_id0, sem_id1, sem_id2, sem_id3,
             sem_t0, sem_t1, sem_t2, sem_a0, sem_a1, sem_a2,
             sem_s0, sem_s1, sem_s2,
             out_acc):
        cid = lax.axis_index("c")
        sid = lax.axis_index("s")
        wid = sid * NC + cid

        idxs = (idxs0, idxs1, idxs2, idxs3)
        idxd = (idxd0, idxd1, idxd2, idxd3)
        tab = (tab0, tab1, tab2)
        ad = (ad0, ad1, ad2)
        sem_is = (sem_is0, sem_is1, sem_is2, sem_is3)
        sem_id = (sem_id0, sem_id1, sem_id2, sem_id3)
        sem_t = (sem_t0, sem_t1, sem_t2)
        sem_a = (sem_a0, sem_a1, sem_a2)
        sem_s = (sem_s0, sem_s1, sem_s2)

        r0 = sid * ROWS
        pltpu.sync_copy(zo_hbm.at[pl.ds(r0, ROWS)], out_acc.at[pl.ds(r0, ROWS)])
        plsc.subcore_barrier()

        ebase = wid * EW

        def issue_idx(j, s):
            pltpu.async_copy(src_hbm.at[pl.ds(ebase + j * B, B)], idxs[s],
                             sem_is[s])
            pltpu.async_copy(dst_hbm.at[pl.ds(ebase + j * B, B)], idxd[s],
                             sem_id[s])

        def wait_idx(j, s):
            pltpu.make_async_copy(src_hbm.at[pl.ds(ebase + j * B, B)],
                                  idxs[s], sem_is[s]).wait()
            pltpu.make_async_copy(dst_hbm.at[pl.ds(ebase + j * B, B)],
                                  idxd[s], sem_id[s]).wait()

        def issue_gathers(b, s):
            pltpu.async_copy(tab_hbm.at[idxs[s]], tab[b], sem_t[b])
            pltpu.async_copy(adt_hbm.at[idxd[s]], ad[b], sem_a[b])

        def wait_gathers(b, s):
            pltpu.make_async_copy(tab_hbm.at[idxs[s]], tab[b], sem_t[b]).wait()
            pltpu.make_async_copy(adt_hbm.at[idxd[s]], ad[b], sem_a[b]).wait()

        dup_idx = lax.rem(lax.iota(jnp.int32, 16), jnp.int32(8))

        def compute(b):
            @plsc.parallel_loop(0, B, step=1, unroll=4)
            def edge_body(e):
                a = tab[b][e, pl.ds(alpha_off, 16)] + ad[b][e, :]
                a = jnp.maximum(a, NEG * a)
                exv = jnp.exp(a)
                if spec == "dup8":
                    exd = exv.at[dup_idx].get(mode="promise_in_bounds")
                    for j in range(8):
                        tab[b][e, pl.ds(16 * j, 16)] = (
                            tab[b][e, pl.ds(16 * j, 16)] * exd)
                else:
                    for lane, start in spec:
                        s_ = exv[lane]
                        tab[b][e, pl.ds(start, 16)] = (
                            tab[b][e, pl.ds(start, 16)] * s_)
                tab[b][e, pl.ds(ex_off, 16)] = exv

        def step(i, k):
            cur = k % 3
            nxt = (k + 1) % 3
            old = (k - 2) % 3
            s_cur, s_nxt, s_pre = k % 4, (k + 1) % 4, (k + 2) % 4
            s_old = (k - 2) % 4
            @pl.when(i >= 2)
            def _():
                pltpu.make_async_copy(
                    tab[old], out_acc.at[idxd[s_old]], sem_s[old]).wait()
            @pl.when(i + 1 < NCH)
            def _():
                wait_idx(i + 1, s_nxt)
                issue_gathers(nxt, s_nxt)
            @pl.when(i + 2 < NCH)
            def _():
                issue_idx(i + 2, s_pre)
            wait_gathers(cur, s_cur)
            compute(cur)
            pltpu.async_copy(tab[cur], out_acc.at[idxd[s_cur]], sem_s[cur],
                             add=True)

        issue_idx(0, 0)
        wait_idx(0, 0)
        issue_gathers(0, 0)
        issue_idx(1, 1)

        def chunk_body(i, carry):
            for k in range(12):
                @pl.when(i % 12 == k)
                def _(k=k):
                    step(i, k)
            return carry

        lax.fori_loop(0, NCH, chunk_body, 0)

        for q in (NCH - 2, NCH - 1):
            pltpu.make_async_copy(tab[q % 3], out_acc.at[idxd[q % 4]],
                                  sem_s[q % 3]).wait()
        plsc.subcore_barrier()

        pltpu.sync_copy(out_acc.at[pl.ds(r0, ROWS)],
                        out_hbm.at[cid, pl.ds(r0, ROWS)])

    return body


def _sc_edge_pass(src, dst, tab, adt, zo, row_w, alpha_off, ex_off, spec):
    mesh = plsc.VectorSubcoreMesh(core_axis_name="c", subcore_axis_name="s",
                                  num_cores=NC, num_subcores=NS)
    body = _make_sc_body(row_w, alpha_off, ex_off, spec)
    dma = pltpu.SemaphoreType.DMA
    return pl.kernel(
        body,
        out_type=jax.ShapeDtypeStruct((NC, NP, row_w), jnp.float32),
        mesh=mesh,
        compiler_params=pltpu.CompilerParams(use_tc_tiling_on_sc=False),
        scratch_types=[
            pltpu.VMEM((B,), jnp.int32),
            pltpu.VMEM((B,), jnp.int32),
            pltpu.VMEM((B,), jnp.int32),
            pltpu.VMEM((B,), jnp.int32),
            pltpu.VMEM((B,), jnp.int32),
            pltpu.VMEM((B,), jnp.int32),
            pltpu.VMEM((B,), jnp.int32),
            pltpu.VMEM((B,), jnp.int32),
            pltpu.VMEM((B, row_w), jnp.float32),
            pltpu.VMEM((B, row_w), jnp.float32),
            pltpu.VMEM((B, row_w), jnp.float32),
            pltpu.VMEM((B, 16), jnp.float32),
            pltpu.VMEM((B, 16), jnp.float32),
            pltpu.VMEM((B, 16), jnp.float32),
            dma, dma, dma, dma, dma, dma, dma, dma, dma,
            dma, dma, dma, dma, dma, dma, dma, dma,
            pltpu.VMEM_SHARED((NP, row_w), jnp.float32),
        ],
    )(src, dst, tab, adt, zo)


def _tc_c(o1_ref, h_ref, ad_ref, w2_ref, as2_ref, ad2_ref,
          b1_ref, p_ref, h2_ref, ado_ref):
    h1 = h_ref[:, 0:128]
    a = h_ref[:, 128:136] + ad_ref[:, 0:8]
    exs = jnp.exp(jnp.where(a >= 0.0, a, NEG * a))
    den = o1_ref[0, :, 128:136] + o1_ref[1, :, 128:136] + exs
    exs_e = jnp.dot(exs, p_ref[:, :], preferred_element_type=jnp.float32)
    den_e = jnp.dot(den, p_ref[:, :], preferred_element_type=jnp.float32)
    num = o1_ref[0, :, 0:128] + o1_ref[1, :, 0:128] + h1 * exs_e
    g = num / (den_e + 1e-16) + b1_ref[0, :][None, :]
    g = jnp.maximum(g, 0.0)
    h2 = jnp.dot(g, w2_ref[:, :], preferred_element_type=jnp.float32)
    asrc2 = jnp.sum(h2 * as2_ref[0, :][None, :], axis=1, keepdims=True)
    adst2 = jnp.sum(h2 * ad2_ref[0, :][None, :], axis=1, keepdims=True)
    z8 = jnp.zeros((RB, 8), jnp.float32)
    z15 = jnp.zeros((RB, 15), jnp.float32)
    h2_ref[:, :] = jnp.concatenate([h2, z8, asrc2, z15], axis=1)
    ado_ref[:, :] = jnp.concatenate([adst2, z15], axis=1)


def _tc_e(o2_ref, h2_ref, ad2_ref, b2_ref, out_ref):
    h2 = h2_ref[:, 0:40]
    a = h2_ref[:, 48:49] + ad2_ref[:, 0:1]
    exs = jnp.exp(jnp.where(a >= 0.0, a, NEG * a))
    den = o2_ref[0, :, 48:49] + o2_ref[1, :, 48:49] + exs
    num = o2_ref[0, :, 0:40] + o2_ref[1, :, 0:40] + h2 * exs
    o = num / (den + 1e-16) + b2_ref[0, :][None, :]
    m = jnp.max(o, axis=1, keepdims=True)
    z = o - m
    lse = jnp.log(jnp.sum(jnp.exp(z), axis=1, keepdims=True))
    out_ref[:, :] = z - lse


def kernel(x, edge_index, W1, att_src1, att_dst1, b1, W2, att_src2, att_dst2, b2):
    src = edge_index[0]
    dst = edge_index[1]

    eye8 = jnp.eye(H, dtype=jnp.float32)
    S1 = (att_src1[:, :, None] * eye8[:, None, :]).reshape(H * K, H)
    T1 = (att_dst1[:, :, None] * eye8[:, None, :]).reshape(H * K, H)
    j = jnp.arange(H * K)
    pvec = (j % H) * K + j // H
    W1t = W1[:, pvec]
    S1t = S1[pvec, :]
    T1t = T1[pvec, :]
    W2t = W2[pvec, :]
    b1t = b1[pvec]
    Pt = (j[None, :] % H == jnp.arange(H)[:, None]).astype(jnp.float32)

    htab1, ad1 = pl.pallas_call(
        _tc_a,
        grid=(N // RB,),
        in_specs=[
            pl.BlockSpec((RB, D), lambda i: (i, 0)),
            pl.BlockSpec((D, D), lambda i: (0, 0)),
            pl.BlockSpec((D, H), lambda i: (0, 0)),
            pl.BlockSpec((D, H), lambda i: (0, 0)),
        ],
        out_specs=[
            pl.BlockSpec((RB, 144), lambda i: (i, 0)),
            pl.BlockSpec((RB, 16), lambda i: (i, 0)),
        ],
        out_shape=[
            jax.ShapeDtypeStruct((N, 144), jnp.float32),
            jax.ShapeDtypeStruct((N, 16), jnp.float32),
        ],
    )(x, W1t, S1t, T1t)

    zo1 = jnp.zeros((NP, 144), jnp.float32)
    out1p = _sc_edge_pass(src, dst, htab1, ad1, zo1,
                          row_w=144, alpha_off=128, ex_off=128, spec="dup8")

    htab2, ad2 = pl.pallas_call(
        _tc_c,
        grid=(N // RB,),
        in_specs=[
            pl.BlockSpec((NC, RB, 144), lambda i: (0, i, 0)),
            pl.BlockSpec((RB, 144), lambda i: (i, 0)),
            pl.BlockSpec((RB, 16), lambda i: (i, 0)),
            pl.BlockSpec((D, C), lambda i: (0, 0)),
            pl.BlockSpec((1, C), lambda i: (0, 0)),
            pl.BlockSpec((1, C), lambda i: (0, 0)),
            pl.BlockSpec((1, D), lambda i: (0, 0)),
            pl.BlockSpec((H, D), lambda i: (0, 0)),
        ],
        out_specs=[
            pl.BlockSpec((RB, 64), lambda i: (i, 0)),
            pl.BlockSpec((RB, 16), lambda i: (i, 0)),
        ],
        out_shape=[
            jax.ShapeDtypeStruct((N, 64), jnp.float32),
            jax.ShapeDtypeStruct((N, 16), jnp.float32),
        ],
    )(out1p, htab1, ad1, W2t, att_src2, att_dst2, b1t.reshape(1, D), Pt)

    zo2 = jnp.zeros((NP, 64), jnp.float32)
    spec2 = [(0, 0), (0, 16), (0, 32)]
    out2p = _sc_edge_pass(src, dst, htab2, ad2, zo2,
                          row_w=64, alpha_off=48, ex_off=48, spec=spec2)

    out = pl.pallas_call(
        _tc_e,
        grid=(N // RB,),
        in_specs=[
            pl.BlockSpec((NC, RB, 64), lambda i: (0, i, 0)),
            pl.BlockSpec((RB, 64), lambda i: (i, 0)),
            pl.BlockSpec((RB, 16), lambda i: (i, 0)),
            pl.BlockSpec((1, C), lambda i: (0, 0)),
        ],
        out_specs=pl.BlockSpec((RB, C), lambda i: (i, 0)),
        out_shape=jax.ShapeDtypeStruct((N, C), jnp.float32),
    )(out2p, htab2, ad2, b2.reshape(1, C))

    return out
